# baseline (speedup 1.0000x reference)
import functools

import jax
import jax.numpy as jnp
from jax import lax
from jax.experimental import pallas as pl
from jax.experimental.pallas import tpu as pltpu
from jax.experimental.pallas import tpu_sc as plsc

ROWS = 16384
COLS = 26
D = 32
B = ROWS * COLS
NC, NS = 2, 16
NW = NC * NS
PER_W = ROWS // NW
CI = 32
NCHUNK = PER_W // CI
CN = CI * COLS

_mesh = plsc.VectorSubcoreMesh(core_axis_name="c", subcore_axis_name="s")

V = 1000000
KA = 800
NCK = V // KA


@functools.partial(
    pl.kernel,
    mesh=_mesh,
    out_type=jax.ShapeDtypeStruct((V, D), jnp.float32),
    scratch_types=[
        pltpu.VMEM((D, KA), jnp.float32),
        pltpu.VMEM((KA, D + 1), jnp.float32),
        pltpu.SemaphoreType.DMA,
        pltpu.SemaphoreType.DMA,
    ],
    compiler_params=pltpu.CompilerParams(
        use_tc_tiling_on_sc=False, needs_layout_passes=False),
)
def _transpose_kernel(tt_hbm, tab_hbm, colbuf, rowbuf, isem, osem):
    wid = lax.axis_index("s") * NC + lax.axis_index("c")
    nck = jnp.where(wid < NCK - (NCK // NW) * NW, NCK // NW + 1, NCK // NW)
    kiota = lax.iota(jnp.int32, 16)

    def chunk_body(t, carry):
        v0 = (wid + t * NW) * KA
        pltpu.async_copy(tt_hbm.at[:, pl.ds(v0, KA)], colbuf, isem).wait()

        def kb_body(kb, c2):
            rvec = kb * 16 + kiota
            for c in range(D):
                vals = colbuf[c, pl.ds(kb * 16, 16)]
                plsc.store_scatter(
                    rowbuf, [rvec, jnp.full((16,), c, jnp.int32)], vals)
            return c2

        lax.fori_loop(0, KA // 16, kb_body, 0)
        pltpu.async_copy(
            rowbuf.at[:, pl.ds(0, D)], tab_hbm.at[pl.ds(v0, KA)], osem).wait()
        return carry

    lax.fori_loop(0, nck, chunk_body, 0)


@functools.partial(
    pl.kernel,
    mesh=_mesh,
    out_type=jax.ShapeDtypeStruct((COLS * D, ROWS), jnp.float32),
    scratch_types=[
        pltpu.VMEM((CN,), jnp.int32),
        pltpu.VMEM((CN,), jnp.int32),
        pltpu.VMEM((CN, D), jnp.float32),
        pltpu.VMEM((CN, D), jnp.float32),
        pltpu.VMEM((COLS * D, CI + 1), jnp.float32),
        pltpu.VMEM((COLS * D, CI + 1), jnp.float32),
        pltpu.SemaphoreType.DMA((2,)),
        pltpu.SemaphoreType.DMA((2,)),
        pltpu.SemaphoreType.DMA((2,)),
    ],
    compiler_params=pltpu.CompilerParams(
        use_tc_tiling_on_sc=False, needs_layout_passes=False),
)
def _gather_kernel(idx_hbm, table_hbm, out_hbm, idx_a, idx_b, rows_a, rows_b,
                   tb_a, tb_b, isem, gsem, osem):
    wid = lax.axis_index("s") * NC + lax.axis_index("c")
    base = wid * PER_W
    idx_v = (idx_a, idx_b)
    rows_v = (rows_a, rows_b)
    tbuf_v = (tb_a, tb_b)

    def idx_copy(j):
        off = (base + j * CI) * COLS
        return pltpu.async_copy(
            idx_hbm.at[pl.ds(off, CN)], idx_v[j % 2], isem.at[j % 2])

    def gather(j):
        return pltpu.async_copy(
            table_hbm.at[idx_v[j % 2]], rows_v[j % 2], gsem.at[j % 2])

    def transpose_chunk(j):
        rv = rows_v[j % 2]
        tb = tbuf_v[j % 2]
        ciota = lax.iota(jnp.int32, 16)

        def i_body(i, carry):
            cvec = jnp.full((16,), 0, jnp.int32) + i
            for jj in range(COLS):
                for half in range(2):
                    vals = rv[i * COLS + jj, pl.ds(half * 16, 16)]
                    rvec = ciota + (jj * D + half * 16)
                    plsc.store_scatter(tb, [rvec, cvec], vals)
            return carry

        lax.fori_loop(0, CI, i_body, 0)

    def writeback(j):
        i0 = base + j * CI
        return pltpu.async_copy(
            tbuf_v[j % 2].at[:, pl.ds(0, CI)], out_hbm.at[:, pl.ds(i0, CI)],
            osem.at[j % 2])

    ih = [None] * NCHUNK
    gh = [None] * NCHUNK
    oh = [None] * NCHUNK
    ih[0] = idx_copy(0)
    ih[0].wait()
    gh[0] = gather(0)
    if NCHUNK > 1:
        ih[1] = idx_copy(1)
    for j in range(NCHUNK):
        gh[j].wait()
        if j + 1 < NCHUNK:
            ih[j + 1].wait()
            gh[j + 1] = gather(j + 1)
            if j + 2 < NCHUNK:
                ih[j + 2] = idx_copy(j + 2)
        if j - 2 >= 0:
            oh[j - 2].wait()
        transpose_chunk(j)
        oh[j] = writeback(j)
    if NCHUNK > 1:
        oh[NCHUNK - 2].wait()
    oh[NCHUNK - 1].wait()


def kernel(indices, table, dummy):
    idx = indices.reshape(-1).astype(jnp.int32)
    tab = _transpose_kernel(table.T)
    o = _gather_kernel(idx, tab)
    return jnp.transpose(o.reshape(COLS, D, ROWS), (2, 0, 1))


# speedup vs baseline: 3.2727x; 3.2727x over previous
import functools

import jax
import jax.numpy as jnp
from jax import lax
from jax.experimental import pallas as pl
from jax.experimental.pallas import tpu as pltpu
from jax.experimental.pallas import tpu_sc as plsc

ROWS = 16384
COLS = 26
D = 32
B = ROWS * COLS
NC, NS = 2, 16
NW = NC * NS
PER_W = ROWS // NW
CI = 32
NCHUNK = PER_W // CI
CN = CI * COLS

_mesh = plsc.VectorSubcoreMesh(core_axis_name="c", subcore_axis_name="s")

V = 1000000
KA = 800
KA4 = KA // 4
NCK = V // KA


@functools.partial(
    pl.kernel,
    mesh=_mesh,
    out_type=jax.ShapeDtypeStruct((V // 4, 4 * D), jnp.float32),
    scratch_types=[
        pltpu.VMEM((KA, D), jnp.float32),
        pltpu.VMEM((KA4, 4 * D), jnp.float32),
        pltpu.SemaphoreType.DMA,
        pltpu.SemaphoreType.DMA,
    ],
    compiler_params=pltpu.CompilerParams(use_tc_tiling_on_sc=True),
)
def _detile_kernel(tab_hbm, out_hbm, vb, vb128, isem, osem):
    wid = lax.axis_index("s") * NC + lax.axis_index("c")
    nck = NCK // NW + 1

    def chunk_body(t, carry):
        ck = wid + t * NW

        @pl.when(ck < NCK)
        def _():
            v0 = pl.multiple_of(ck * KA, KA)
            pltpu.async_copy(tab_hbm.at[pl.ds(v0, KA)], vb, isem).wait()

            def q_body(q, c2):
                for m in range(4):
                    for half in range(2):
                        vb128[q, pl.ds(m * D + half * 16, 16)] = (
                            vb[q * 4 + m, pl.ds(half * 16, 16)])
                return c2

            lax.fori_loop(0, KA4, q_body, 0)
            pltpu.async_copy(
                vb128, out_hbm.at[pl.ds(pl.multiple_of(v0 // 4, KA4), KA4)],
                osem).wait()

        return carry

    lax.fori_loop(0, nck, chunk_body, 0)


@functools.partial(
    pl.kernel,
    mesh=_mesh,
    out_type=jax.ShapeDtypeStruct((COLS * D, ROWS), jnp.float32),
    scratch_types=[
        pltpu.VMEM((CN,), jnp.int32),
        pltpu.VMEM((CN,), jnp.int32),
        pltpu.VMEM((CN, D), jnp.float32),
        pltpu.VMEM((CN, D), jnp.float32),
        pltpu.VMEM((COLS * D, CI + 1), jnp.float32),
        pltpu.VMEM((COLS * D, CI + 1), jnp.float32),
        pltpu.SemaphoreType.DMA((2,)),
        pltpu.SemaphoreType.DMA((2,)),
        pltpu.SemaphoreType.DMA((2,)),
    ],
    compiler_params=pltpu.CompilerParams(
        use_tc_tiling_on_sc=False, needs_layout_passes=False),
)
def _gather_kernel(idx_hbm, table_hbm, out_hbm, idx_a, idx_b, rows_a, rows_b,
                   tb_a, tb_b, isem, gsem, osem):
    wid = lax.axis_index("s") * NC + lax.axis_index("c")
    base = wid * PER_W
    idx_v = (idx_a, idx_b)
    rows_v = (rows_a, rows_b)
    tbuf_v = (tb_a, tb_b)

    def idx_copy(j):
        off = (base + j * CI) * COLS
        return pltpu.async_copy(
            idx_hbm.at[pl.ds(off, CN)], idx_v[j % 2], isem.at[j % 2])

    def gather(j):
        return pltpu.async_copy(
            table_hbm.at[idx_v[j % 2]], rows_v[j % 2], gsem.at[j % 2])

    def transpose_chunk(j):
        rv = rows_v[j % 2]
        tb = tbuf_v[j % 2]
        ciota = lax.iota(jnp.int32, 16)

        def i_body(i, carry):
            cvec = jnp.full((16,), 0, jnp.int32) + i
            for jj in range(COLS):
                for half in range(2):
                    vals = rv[i * COLS + jj, pl.ds(half * 16, 16)]
                    rvec = ciota + (jj * D + half * 16)
                    plsc.store_scatter(tb, [rvec, cvec], vals)
            return carry

        lax.fori_loop(0, CI, i_body, 0)

    def writeback(j):
        i0 = base + j * CI
        return pltpu.async_copy(
            tbuf_v[j % 2].at[:, pl.ds(0, CI)], out_hbm.at[:, pl.ds(i0, CI)],
            osem.at[j % 2])

    ih = [None] * NCHUNK
    gh = [None] * NCHUNK
    oh = [None] * NCHUNK
    ih[0] = idx_copy(0)
    ih[0].wait()
    gh[0] = gather(0)
    if NCHUNK > 1:
        ih[1] = idx_copy(1)
    for j in range(NCHUNK):
        gh[j].wait()
        if j + 1 < NCHUNK:
            ih[j + 1].wait()
            gh[j + 1] = gather(j + 1)
            if j + 2 < NCHUNK:
                ih[j + 2] = idx_copy(j + 2)
        if j - 2 >= 0:
            oh[j - 2].wait()
        transpose_chunk(j)
        oh[j] = writeback(j)
    if NCHUNK > 1:
        oh[NCHUNK - 2].wait()
    oh[NCHUNK - 1].wait()


def kernel(indices, table, dummy):
    idx = indices.reshape(-1).astype(jnp.int32)
    tab = _detile_kernel(table).reshape(V, D)
    o = _gather_kernel(idx, tab)
    return jnp.transpose(o.reshape(COLS, D, ROWS), (2, 0, 1))


# speedup vs baseline: 4.5060x; 1.3769x over previous
import functools

import jax
import jax.numpy as jnp
from jax import lax
from jax.experimental import pallas as pl
from jax.experimental.pallas import tpu as pltpu
from jax.experimental.pallas import tpu_sc as plsc

ROWS = 16384
COLS = 26
D = 32
B = ROWS * COLS
NC, NS = 2, 16
NW = NC * NS
PER_W = ROWS // NW
CI = 32
NCHUNK = PER_W // CI
CN = CI * COLS

_mesh = plsc.VectorSubcoreMesh(core_axis_name="c", subcore_axis_name="s")

@functools.partial(
    pl.kernel,
    mesh=_mesh,
    out_type=jax.ShapeDtypeStruct((COLS * D, ROWS), jnp.float32),
    scratch_types=[
        pltpu.VMEM((CN,), jnp.int32),
        pltpu.VMEM((CN,), jnp.int32),
        pltpu.VMEM((CN, D), jnp.float32),
        pltpu.VMEM((CN, D), jnp.float32),
        pltpu.VMEM((COLS * D, CI + 1), jnp.float32),
        pltpu.VMEM((COLS * D, CI + 1), jnp.float32),
        pltpu.SemaphoreType.DMA((2,)),
        pltpu.SemaphoreType.DMA((2,)),
        pltpu.SemaphoreType.DMA((2,)),
    ],
    compiler_params=pltpu.CompilerParams(
        use_tc_tiling_on_sc=False, needs_layout_passes=False),
)
def _gather_kernel(idx_hbm, table_hbm, out_hbm, idx_a, idx_b, rows_a, rows_b,
                   tb_a, tb_b, isem, gsem, osem):
    wid = lax.axis_index("s") * NC + lax.axis_index("c")
    base = wid * PER_W
    idx_v = (idx_a, idx_b)
    rows_v = (rows_a, rows_b)
    tbuf_v = (tb_a, tb_b)

    def idx_copy(j):
        off = (base + j * CI) * COLS
        return pltpu.async_copy(
            idx_hbm.at[pl.ds(off, CN)], idx_v[j % 2], isem.at[j % 2])

    def gather(j):
        return pltpu.async_copy(
            table_hbm.at[idx_v[j % 2]], rows_v[j % 2], gsem.at[j % 2])

    def transpose_chunk(j):
        rv = rows_v[j % 2]
        tb = tbuf_v[j % 2]
        ciota = lax.iota(jnp.int32, 16)

        def i_body(i, carry):
            cvec = jnp.full((16,), 0, jnp.int32) + i
            for jj in range(COLS):
                for half in range(2):
                    vals = rv[i * COLS + jj, pl.ds(half * 16, 16)]
                    rvec = ciota + (jj * D + half * 16)
                    plsc.store_scatter(tb, [rvec, cvec], vals)
            return carry

        lax.fori_loop(0, CI, i_body, 0)

    def writeback(j):
        i0 = base + j * CI
        return pltpu.async_copy(
            tbuf_v[j % 2].at[:, pl.ds(0, CI)], out_hbm.at[:, pl.ds(i0, CI)],
            osem.at[j % 2])

    ih = [None] * NCHUNK
    gh = [None] * NCHUNK
    oh = [None] * NCHUNK
    ih[0] = idx_copy(0)
    ih[0].wait()
    gh[0] = gather(0)
    if NCHUNK > 1:
        ih[1] = idx_copy(1)
    for j in range(NCHUNK):
        gh[j].wait()
        if j + 1 < NCHUNK:
            ih[j + 1].wait()
            gh[j + 1] = gather(j + 1)
            if j + 2 < NCHUNK:
                ih[j + 2] = idx_copy(j + 2)
        if j - 2 >= 0:
            oh[j - 2].wait()
        transpose_chunk(j)
        oh[j] = writeback(j)
    if NCHUNK > 1:
        oh[NCHUNK - 2].wait()
    oh[NCHUNK - 1].wait()


def kernel(indices, table, dummy):
    idx = indices.reshape(-1).astype(jnp.int32)
    o = _gather_kernel(idx, table)
    return jnp.transpose(o.reshape(COLS, D, ROWS), (2, 0, 1))


# speedup vs baseline: 4.5147x; 1.0019x over previous
import functools

import jax
import jax.numpy as jnp
from jax import lax
from jax.experimental import pallas as pl
from jax.experimental.pallas import tpu as pltpu
from jax.experimental.pallas import tpu_sc as plsc

ROWS = 16384
COLS = 26
D = 32
B = ROWS * COLS
NC, NS = 2, 16
NW = NC * NS
PER_W = ROWS // NW
CI = 32
NCHUNK = PER_W // CI
CN = CI * COLS

_mesh = plsc.VectorSubcoreMesh(core_axis_name="c", subcore_axis_name="s")

@functools.partial(
    pl.kernel,
    mesh=_mesh,
    out_type=jax.ShapeDtypeStruct((COLS * D, ROWS), jnp.float32),
    scratch_types=[
        pltpu.VMEM((CN,), jnp.int32),
        pltpu.VMEM((CN,), jnp.int32),
        pltpu.VMEM((CN, D), jnp.float32),
        pltpu.VMEM((CN, D), jnp.float32),
        pltpu.VMEM((COLS * D, CI + 1), jnp.float32),
        pltpu.VMEM((COLS * D, CI + 1), jnp.float32),
        pltpu.SemaphoreType.DMA((2,)),
        pltpu.SemaphoreType.DMA((2,)),
        pltpu.SemaphoreType.DMA((2,)),
    ],
    compiler_params=pltpu.CompilerParams(
        use_tc_tiling_on_sc=False, needs_layout_passes=False),
)
def _gather_kernel(idx_hbm, table_hbm, out_hbm, idx_a, idx_b, rows_a, rows_b,
                   tb_a, tb_b, isem, gsem, osem):
    wid = lax.axis_index("s") * NC + lax.axis_index("c")
    base = wid * PER_W
    idx_v = (idx_a, idx_b)
    rows_v = (rows_a, rows_b)
    tbuf_v = (tb_a, tb_b)

    def idx_copy(j):
        off = (base + j * CI) * COLS
        return pltpu.async_copy(
            idx_hbm.at[pl.ds(off, CN)], idx_v[j % 2], isem.at[j % 2])

    def gather(j):
        return pltpu.async_copy(
            table_hbm.at[idx_v[j % 2]], rows_v[j % 2], gsem.at[j % 2])

    def transpose_chunk(j):
        rv = rows_v[j % 2]
        tb = tbuf_v[j % 2]
        ciota = lax.iota(jnp.int32, 16)

        def i_body(i2, carry):
            for u in range(2):
                i = i2 * 2 + u
                cvec = jnp.full((16,), 0, jnp.int32) + i
                for jj in range(COLS):
                    for half in range(2):
                        vals = rv[i * COLS + jj, pl.ds(half * 16, 16)]
                        rvec = ciota + (jj * D + half * 16)
                        plsc.store_scatter(tb, [rvec, cvec], vals)
            return carry

        lax.fori_loop(0, CI // 2, i_body, 0)

    def writeback(j):
        i0 = base + j * CI
        return pltpu.async_copy(
            tbuf_v[j % 2].at[:, pl.ds(0, CI)], out_hbm.at[:, pl.ds(i0, CI)],
            osem.at[j % 2])

    ih = [None] * NCHUNK
    gh = [None] * NCHUNK
    oh = [None] * NCHUNK
    ih[0] = idx_copy(0)
    ih[0].wait()
    gh[0] = gather(0)
    if NCHUNK > 1:
        ih[1] = idx_copy(1)
    for j in range(NCHUNK):
        gh[j].wait()
        if j + 1 < NCHUNK:
            ih[j + 1].wait()
            gh[j + 1] = gather(j + 1)
            if j + 2 < NCHUNK:
                ih[j + 2] = idx_copy(j + 2)
        if j - 2 >= 0:
            oh[j - 2].wait()
        transpose_chunk(j)
        oh[j] = writeback(j)
    if NCHUNK > 1:
        oh[NCHUNK - 2].wait()
    oh[NCHUNK - 1].wait()


def kernel(indices, table, dummy):
    idx = indices.reshape(-1).astype(jnp.int32)
    o = _gather_kernel(idx, table)
    return jnp.transpose(o.reshape(COLS, D, ROWS), (2, 0, 1))
